# baseline (speedup 1.0000x reference)
import jax
import jax.numpy as jnp
from jax import lax
from jax.experimental import pallas as pl
from jax.experimental.pallas import tpu as pltpu
from jax.experimental.pallas import tpu_sc as plsc

_B = 16384
_L = 200
_EMB = 64
_VOCAB = 1000000
_NC, _NS = 2, 16
_NW = _NC * _NS
_ROWS_PER_W = _B // _NW
_CA, _CB = 96, 104
_G = 64
_GROUPS = _ROWS_PER_W // _G
_INV_L = 1.0 / _L


_MROWS = 2000


def _lin_body(in_ref, out_ref):
    x = in_ref[...]
    a = x[0:_MROWS // 2, :]
    b = x[_MROWS // 2:, :]
    out_ref[...] = jnp.concatenate([a, b], axis=1)


@jax.jit
def _linearize(table):
    grid = _VOCAB // _MROWS
    return pl.pallas_call(
        _lin_body,
        grid=(grid,),
        in_specs=[pl.BlockSpec((_MROWS, _EMB), lambda i: (i, 0))],
        out_specs=pl.BlockSpec((_MROWS // 2, 2 * _EMB), lambda i: (i, 0)),
        out_shape=jax.ShapeDtypeStruct((_VOCAB // 2, 2 * _EMB), jnp.float32),
    )(table)


def _pool_body(idx_hbm, table_hbm, out_hbm, idx_a, idx_b, idx_at, idx_bt,
               buf0, buf1, buf2, buf3, out_v, sem0, sem1, sem2, sem3):
    wid = lax.axis_index("s") * _NC + lax.axis_index("c")
    row0 = wid * _ROWS_PER_W
    half = _MROWS // 2

    def remap(v):
        g = v // _MROWS
        r = v - g * _MROWS
        h = jnp.where(r >= half, jnp.int32(1), jnp.int32(0))
        return _MROWS * g + 2 * (r - half * h) + h

    def xform(src, dst, offs):
        def body(rr, _):
            for k in offs:
                dst[rr, pl.ds(k, 16)] = src[rr, pl.ds(k, 16)]
            return 0
        lax.fori_loop(0, _G, body, 0)

    def accum(buf, n):
        def body(j, accs):
            a0, a1, a2, a3 = accs
            a0 = a0 + buf[j, 0:16]
            a1 = a1 + buf[j, 16:32]
            a2 = a2 + buf[j, 32:48]
            a3 = a3 + buf[j, 48:64]
            return (a0, a1, a2, a3)
        z = jnp.zeros((16,), jnp.float32)
        return lax.fori_loop(0, n, body, (z, z, z, z), unroll=4)

    def group(g, _):
        r0 = row0 + g * _G
        pltpu.sync_copy(idx_hbm.at[pl.ds(r0, _G), pl.ds(0, _CA)], idx_a)
        pltpu.sync_copy(idx_hbm.at[pl.ds(r0, _G), pl.ds(_CA, _CB)], idx_b)
        xform(idx_a, idx_at, (0, 16, 32, 48, 64, 80))
        xform(idx_b, idx_bt, (0, 16, 32, 48, 64, 80, 88))
        pltpu.async_copy(table_hbm.at[idx_at.at[0]], buf0, sem0)
        pltpu.async_copy(table_hbm.at[idx_bt.at[0]], buf1, sem1)
        pltpu.async_copy(table_hbm.at[idx_at.at[1]], buf2, sem2)
        pltpu.async_copy(table_hbm.at[idx_bt.at[1]], buf3, sem3)

        def pair(p, _):
            ra = 2 * p
            rb = 2 * p + 1

            pltpu.make_async_copy(table_hbm.at[idx_at.at[0]], buf0, sem0).wait()
            a0, a1, a2, a3 = accum(buf0, _CA)

            @pl.when(ra + 2 < _G)
            def _():
                pltpu.async_copy(table_hbm.at[idx_at.at[ra + 2]], buf0, sem0)

            pltpu.make_async_copy(table_hbm.at[idx_bt.at[0]], buf1, sem1).wait()
            b0, b1, b2, b3 = accum(buf1, _CB)
            out_v[ra, 0:16] = a0 + b0
            out_v[ra, 16:32] = a1 + b1
            out_v[ra, 32:48] = a2 + b2
            out_v[ra, 48:64] = a3 + b3

            @pl.when(ra + 2 < _G)
            def _():
                pltpu.async_copy(table_hbm.at[idx_bt.at[ra + 2]], buf1, sem1)

            pltpu.make_async_copy(table_hbm.at[idx_at.at[0]], buf2, sem2).wait()
            a0, a1, a2, a3 = accum(buf2, _CA)

            @pl.when(rb + 2 < _G)
            def _():
                pltpu.async_copy(table_hbm.at[idx_at.at[rb + 2]], buf2, sem2)

            pltpu.make_async_copy(table_hbm.at[idx_bt.at[0]], buf3, sem3).wait()
            b0, b1, b2, b3 = accum(buf3, _CB)
            out_v[rb, 0:16] = a0 + b0
            out_v[rb, 16:32] = a1 + b1
            out_v[rb, 32:48] = a2 + b2
            out_v[rb, 48:64] = a3 + b3

            @pl.when(rb + 2 < _G)
            def _():
                pltpu.async_copy(table_hbm.at[idx_bt.at[rb + 2]], buf3, sem3)

            return 0

        lax.fori_loop(0, _G // 2, pair, 0)
        pltpu.sync_copy(out_v, out_hbm.at[pl.ds(r0, _G), :])
        return 0

    lax.fori_loop(0, _GROUPS, group, 0)


@jax.jit
def _pool(idx, table_hbm):
    mesh = plsc.VectorSubcoreMesh(core_axis_name="c", subcore_axis_name="s")
    return pl.kernel(
        _pool_body,
        out_type=jax.ShapeDtypeStruct((_B, _EMB), jnp.float32),
        mesh=mesh,
        compiler_params=pltpu.CompilerParams(use_tc_tiling_on_sc=False),
        scratch_types=[
            pltpu.VMEM((_G, _CA), jnp.int32),
            pltpu.VMEM((_G, _CB), jnp.int32),
            pltpu.VMEM((_G, _CA), jnp.int32),
            pltpu.VMEM((_G, _CB), jnp.int32),
            pltpu.VMEM((_CA, _EMB), jnp.float32),
            pltpu.VMEM((_CB, _EMB), jnp.float32),
            pltpu.VMEM((_CA, _EMB), jnp.float32),
            pltpu.VMEM((_CB, _EMB), jnp.float32),
            pltpu.VMEM((_G, _EMB), jnp.float32),
            pltpu.SemaphoreType.DMA,
            pltpu.SemaphoreType.DMA,
            pltpu.SemaphoreType.DMA,
            pltpu.SemaphoreType.DMA,
        ],
    )(idx, table_hbm)


_BLK = 256


def _mlp_body(pool_ref, sem_ref, wph_ref, bph_ref, wse_ref, bse_ref,
              wc1_ref, wc2_ref, bc_ref, wo_ref, bo_ref, out_ref):
    pool = pool_ref[...] * _INV_L
    dn = (((1,), (1,)), ((), ()))
    ph = lax.dot_general(pool, wph_ref[...], dn,
                         preferred_element_type=jnp.float32) + bph_ref[...]
    se = lax.dot_general(sem_ref[...], wse_ref[...], dn,
                         preferred_element_type=jnp.float32) + bse_ref[...]
    x = (lax.dot_general(ph, wc1_ref[...], dn,
                         preferred_element_type=jnp.float32)
         + lax.dot_general(se, wc2_ref[...], dn,
                           preferred_element_type=jnp.float32)
         + bc_ref[...])
    x = jnp.maximum(x, 0.0)
    o = jnp.sum(x * wo_ref[...], axis=1, keepdims=True) + bo_ref[...]
    out_ref[...] = jax.nn.sigmoid(o)


@jax.jit
def _mlp(pooled, semantic, W_ph, b_ph, W_se, b_se, wc1, wc2, b_c, W_o, b_o):
    n_blk = _B // _BLK
    full = lambda shape: pl.BlockSpec(shape, lambda i: (0, 0))
    return pl.pallas_call(
        _mlp_body,
        grid=(n_blk,),
        in_specs=[
            pl.BlockSpec((_BLK, _EMB), lambda i: (i, 0)),
            pl.BlockSpec((_BLK, 256), lambda i: (i, 0)),
            full((128, _EMB)),
            full((1, 128)),
            full((128, 256)),
            full((1, 128)),
            full((64, 128)),
            full((64, 128)),
            full((1, 64)),
            full((1, 64)),
            full((1, 1)),
        ],
        out_specs=pl.BlockSpec((_BLK, 1), lambda i: (i, 0)),
        out_shape=jax.ShapeDtypeStruct((_B, 1), jnp.float32),
    )(pooled, semantic, W_ph, b_ph, W_se, b_se, wc1, wc2, b_c, W_o, b_o)


def kernel(phonetic_token, semantic_embed, emb_table,
           W_ph, b_ph, W_se, b_se, W_c, b_c, W_o, b_o):
    table_lin = _linearize(emb_table)
    pooled = _pool(phonetic_token.astype(jnp.int32),
                   table_lin.reshape(_VOCAB, _EMB))
    return _mlp(pooled, semantic_embed,
                W_ph, b_ph.reshape(1, -1),
                W_se, b_se.reshape(1, -1),
                W_c[:, :128], W_c[:, 128:], b_c.reshape(1, -1),
                W_o, b_o.reshape(1, -1))


# speedup vs baseline: 1.1835x; 1.1835x over previous
import jax
import jax.numpy as jnp
from jax import lax
from jax.experimental import pallas as pl
from jax.experimental.pallas import tpu as pltpu
from jax.experimental.pallas import tpu_sc as plsc

_B = 16384
_L = 200
_EMB = 64
_VOCAB = 1000000
_NC, _NS = 2, 16
_NW = _NC * _NS
_ROWS_PER_W = _B // _NW
_CA, _CB = 96, 104
_G = 64
_GROUPS = _ROWS_PER_W // _G
_INV_L = 1.0 / _L


def _pool_body(idx_hbm, table_hbm, out_hbm, idx_a, idx_b,
               buf0, buf1, buf2, buf3, out_v, sem0, sem1, sem2, sem3):
    wid = lax.axis_index("s") * _NC + lax.axis_index("c")
    row0 = wid * _ROWS_PER_W

    def accum(buf, n):
        def body(j, accs):
            a0, a1, a2, a3 = accs
            a0 = a0 + buf[j, 0:16]
            a1 = a1 + buf[j, 16:32]
            a2 = a2 + buf[j, 32:48]
            a3 = a3 + buf[j, 48:64]
            return (a0, a1, a2, a3)
        z = jnp.zeros((16,), jnp.float32)
        return lax.fori_loop(0, n, body, (z, z, z, z), unroll=4)

    def group(g, _):
        r0 = row0 + g * _G
        pltpu.sync_copy(idx_hbm.at[pl.ds(r0, _G), pl.ds(0, _CA)], idx_a)
        pltpu.sync_copy(idx_hbm.at[pl.ds(r0, _G), pl.ds(_CA, _CB)], idx_b)
        pltpu.async_copy(table_hbm.at[idx_a.at[0]], buf0, sem0)
        pltpu.async_copy(table_hbm.at[idx_b.at[0]], buf1, sem1)
        pltpu.async_copy(table_hbm.at[idx_a.at[1]], buf2, sem2)
        pltpu.async_copy(table_hbm.at[idx_b.at[1]], buf3, sem3)

        def pair(p, _):
            ra = 2 * p
            rb = 2 * p + 1

            pltpu.make_async_copy(table_hbm.at[idx_a.at[0]], buf0, sem0).wait()
            a0, a1, a2, a3 = accum(buf0, _CA)

            @pl.when(ra + 2 < _G)
            def _():
                pltpu.async_copy(table_hbm.at[idx_a.at[ra + 2]], buf0, sem0)

            pltpu.make_async_copy(table_hbm.at[idx_b.at[0]], buf1, sem1).wait()
            b0, b1, b2, b3 = accum(buf1, _CB)
            out_v[ra, 0:16] = a0 + b0
            out_v[ra, 16:32] = a1 + b1
            out_v[ra, 32:48] = a2 + b2
            out_v[ra, 48:64] = a3 + b3

            @pl.when(ra + 2 < _G)
            def _():
                pltpu.async_copy(table_hbm.at[idx_b.at[ra + 2]], buf1, sem1)

            pltpu.make_async_copy(table_hbm.at[idx_a.at[0]], buf2, sem2).wait()
            a0, a1, a2, a3 = accum(buf2, _CA)

            @pl.when(rb + 2 < _G)
            def _():
                pltpu.async_copy(table_hbm.at[idx_a.at[rb + 2]], buf2, sem2)

            pltpu.make_async_copy(table_hbm.at[idx_b.at[0]], buf3, sem3).wait()
            b0, b1, b2, b3 = accum(buf3, _CB)
            out_v[rb, 0:16] = a0 + b0
            out_v[rb, 16:32] = a1 + b1
            out_v[rb, 32:48] = a2 + b2
            out_v[rb, 48:64] = a3 + b3

            @pl.when(rb + 2 < _G)
            def _():
                pltpu.async_copy(table_hbm.at[idx_b.at[rb + 2]], buf3, sem3)

            return 0

        lax.fori_loop(0, _G // 2, pair, 0)
        pltpu.sync_copy(out_v, out_hbm.at[pl.ds(r0, _G), :])
        return 0

    lax.fori_loop(0, _GROUPS, group, 0)


@jax.jit
def _pool(idx, table_hbm):
    mesh = plsc.VectorSubcoreMesh(core_axis_name="c", subcore_axis_name="s")
    return pl.kernel(
        _pool_body,
        out_type=jax.ShapeDtypeStruct((_B, _EMB), jnp.float32),
        mesh=mesh,
        compiler_params=pltpu.CompilerParams(use_tc_tiling_on_sc=False),
        scratch_types=[
            pltpu.VMEM((_G, _CA), jnp.int32),
            pltpu.VMEM((_G, _CB), jnp.int32),
            pltpu.VMEM((_CA, _EMB), jnp.float32),
            pltpu.VMEM((_CB, _EMB), jnp.float32),
            pltpu.VMEM((_CA, _EMB), jnp.float32),
            pltpu.VMEM((_CB, _EMB), jnp.float32),
            pltpu.VMEM((_G, _EMB), jnp.float32),
            pltpu.SemaphoreType.DMA,
            pltpu.SemaphoreType.DMA,
            pltpu.SemaphoreType.DMA,
            pltpu.SemaphoreType.DMA,
        ],
    )(idx, table_hbm)


_BLK = 256


def _mlp_body(pool_ref, sem_ref, wph_ref, bph_ref, wse_ref, bse_ref,
              wc1_ref, wc2_ref, bc_ref, wo_ref, bo_ref, out_ref):
    pool = pool_ref[...] * _INV_L
    dn = (((1,), (1,)), ((), ()))
    ph = lax.dot_general(pool, wph_ref[...], dn,
                         preferred_element_type=jnp.float32) + bph_ref[...]
    se = lax.dot_general(sem_ref[...], wse_ref[...], dn,
                         preferred_element_type=jnp.float32) + bse_ref[...]
    x = (lax.dot_general(ph, wc1_ref[...], dn,
                         preferred_element_type=jnp.float32)
         + lax.dot_general(se, wc2_ref[...], dn,
                           preferred_element_type=jnp.float32)
         + bc_ref[...])
    x = jnp.maximum(x, 0.0)
    o = jnp.sum(x * wo_ref[...], axis=1, keepdims=True) + bo_ref[...]
    out_ref[...] = jax.nn.sigmoid(o)


@jax.jit
def _mlp(pooled, semantic, W_ph, b_ph, W_se, b_se, wc1, wc2, b_c, W_o, b_o):
    n_blk = _B // _BLK
    full = lambda shape: pl.BlockSpec(shape, lambda i: (0, 0))
    return pl.pallas_call(
        _mlp_body,
        grid=(n_blk,),
        in_specs=[
            pl.BlockSpec((_BLK, _EMB), lambda i: (i, 0)),
            pl.BlockSpec((_BLK, 256), lambda i: (i, 0)),
            full((128, _EMB)),
            full((1, 128)),
            full((128, 256)),
            full((1, 128)),
            full((64, 128)),
            full((64, 128)),
            full((1, 64)),
            full((1, 64)),
            full((1, 1)),
        ],
        out_specs=pl.BlockSpec((_BLK, 1), lambda i: (i, 0)),
        out_shape=jax.ShapeDtypeStruct((_B, 1), jnp.float32),
    )(pooled, semantic, W_ph, b_ph, W_se, b_se, wc1, wc2, b_c, W_o, b_o)


def kernel(phonetic_token, semantic_embed, emb_table,
           W_ph, b_ph, W_se, b_se, W_c, b_c, W_o, b_o):
    t2 = lax.optimization_barrier(emb_table.reshape(_VOCAB // 2, 2 * _EMB))
    pooled = _pool(phonetic_token.astype(jnp.int32),
                   t2.reshape(_VOCAB, _EMB))
    return _mlp(pooled, semantic_embed,
                W_ph, b_ph.reshape(1, -1),
                W_se, b_se.reshape(1, -1),
                W_c[:, :128], W_c[:, 128:], b_c.reshape(1, -1),
                W_o, b_o.reshape(1, -1))


# speedup vs baseline: 1.1958x; 1.0104x over previous
import jax
import jax.numpy as jnp
from jax import lax
from jax.experimental import pallas as pl
from jax.experimental.pallas import tpu as pltpu
from jax.experimental.pallas import tpu_sc as plsc

_B = 16384
_L = 200
_EMB = 64
_VOCAB = 1000000
_NC, _NS = 2, 16
_NW = _NC * _NS
_ROWS_PER_W = _B // _NW
_CA, _CB = 96, 104
_G = 128
_GROUPS = _ROWS_PER_W // _G
_INV_L = 1.0 / _L


def _pool_body(idx_hbm, table_hbm, out_hbm, idx_a, idx_b,
               buf0, buf1, buf2, buf3, out_v, sem0, sem1, sem2, sem3):
    wid = lax.axis_index("s") * _NC + lax.axis_index("c")
    row0 = wid * _ROWS_PER_W

    def accum(buf, n):
        def body(j, accs):
            a0, a1, a2, a3 = accs
            a0 = a0 + buf[j, 0:16]
            a1 = a1 + buf[j, 16:32]
            a2 = a2 + buf[j, 32:48]
            a3 = a3 + buf[j, 48:64]
            return (a0, a1, a2, a3)
        z = jnp.zeros((16,), jnp.float32)
        return lax.fori_loop(0, n, body, (z, z, z, z), unroll=8)

    def group(g, _):
        r0 = row0 + g * _G
        pltpu.sync_copy(idx_hbm.at[pl.ds(r0, _G), pl.ds(0, _CA)], idx_a)
        pltpu.sync_copy(idx_hbm.at[pl.ds(r0, _G), pl.ds(_CA, _CB)], idx_b)
        pltpu.async_copy(table_hbm.at[idx_a.at[0]], buf0, sem0)
        pltpu.async_copy(table_hbm.at[idx_b.at[0]], buf1, sem1)
        pltpu.async_copy(table_hbm.at[idx_a.at[1]], buf2, sem2)
        pltpu.async_copy(table_hbm.at[idx_b.at[1]], buf3, sem3)

        def pair(p, _):
            ra = 2 * p
            rb = 2 * p + 1

            pltpu.make_async_copy(table_hbm.at[idx_a.at[0]], buf0, sem0).wait()
            a0, a1, a2, a3 = accum(buf0, _CA)

            @pl.when(ra + 2 < _G)
            def _():
                pltpu.async_copy(table_hbm.at[idx_a.at[ra + 2]], buf0, sem0)

            pltpu.make_async_copy(table_hbm.at[idx_b.at[0]], buf1, sem1).wait()
            b0, b1, b2, b3 = accum(buf1, _CB)
            out_v[ra, 0:16] = a0 + b0
            out_v[ra, 16:32] = a1 + b1
            out_v[ra, 32:48] = a2 + b2
            out_v[ra, 48:64] = a3 + b3

            @pl.when(ra + 2 < _G)
            def _():
                pltpu.async_copy(table_hbm.at[idx_b.at[ra + 2]], buf1, sem1)

            pltpu.make_async_copy(table_hbm.at[idx_a.at[0]], buf2, sem2).wait()
            a0, a1, a2, a3 = accum(buf2, _CA)

            @pl.when(rb + 2 < _G)
            def _():
                pltpu.async_copy(table_hbm.at[idx_a.at[rb + 2]], buf2, sem2)

            pltpu.make_async_copy(table_hbm.at[idx_b.at[0]], buf3, sem3).wait()
            b0, b1, b2, b3 = accum(buf3, _CB)
            out_v[rb, 0:16] = a0 + b0
            out_v[rb, 16:32] = a1 + b1
            out_v[rb, 32:48] = a2 + b2
            out_v[rb, 48:64] = a3 + b3

            @pl.when(rb + 2 < _G)
            def _():
                pltpu.async_copy(table_hbm.at[idx_b.at[rb + 2]], buf3, sem3)

            return 0

        lax.fori_loop(0, _G // 2, pair, 0)
        pltpu.sync_copy(out_v, out_hbm.at[pl.ds(r0, _G), :])
        return 0

    lax.fori_loop(0, _GROUPS, group, 0)


@jax.jit
def _pool(idx, table_hbm):
    mesh = plsc.VectorSubcoreMesh(core_axis_name="c", subcore_axis_name="s")
    return pl.kernel(
        _pool_body,
        out_type=jax.ShapeDtypeStruct((_B, _EMB), jnp.float32),
        mesh=mesh,
        compiler_params=pltpu.CompilerParams(use_tc_tiling_on_sc=False),
        scratch_types=[
            pltpu.VMEM((_G, _CA), jnp.int32),
            pltpu.VMEM((_G, _CB), jnp.int32),
            pltpu.VMEM((_CA, _EMB), jnp.float32),
            pltpu.VMEM((_CB, _EMB), jnp.float32),
            pltpu.VMEM((_CA, _EMB), jnp.float32),
            pltpu.VMEM((_CB, _EMB), jnp.float32),
            pltpu.VMEM((_G, _EMB), jnp.float32),
            pltpu.SemaphoreType.DMA,
            pltpu.SemaphoreType.DMA,
            pltpu.SemaphoreType.DMA,
            pltpu.SemaphoreType.DMA,
        ],
    )(idx, table_hbm)


_BLK = 256


def _mlp_body(pool_ref, sem_ref, wph_ref, bph_ref, wse_ref, bse_ref,
              wc1_ref, wc2_ref, bc_ref, wo_ref, bo_ref, out_ref):
    pool = pool_ref[...] * _INV_L
    dn = (((1,), (1,)), ((), ()))
    ph = lax.dot_general(pool, wph_ref[...], dn,
                         preferred_element_type=jnp.float32) + bph_ref[...]
    se = lax.dot_general(sem_ref[...], wse_ref[...], dn,
                         preferred_element_type=jnp.float32) + bse_ref[...]
    x = (lax.dot_general(ph, wc1_ref[...], dn,
                         preferred_element_type=jnp.float32)
         + lax.dot_general(se, wc2_ref[...], dn,
                           preferred_element_type=jnp.float32)
         + bc_ref[...])
    x = jnp.maximum(x, 0.0)
    o = jnp.sum(x * wo_ref[...], axis=1, keepdims=True) + bo_ref[...]
    out_ref[...] = jax.nn.sigmoid(o)


@jax.jit
def _mlp(pooled, semantic, W_ph, b_ph, W_se, b_se, wc1, wc2, b_c, W_o, b_o):
    n_blk = _B // _BLK
    full = lambda shape: pl.BlockSpec(shape, lambda i: (0, 0))
    return pl.pallas_call(
        _mlp_body,
        grid=(n_blk,),
        in_specs=[
            pl.BlockSpec((_BLK, _EMB), lambda i: (i, 0)),
            pl.BlockSpec((_BLK, 256), lambda i: (i, 0)),
            full((128, _EMB)),
            full((1, 128)),
            full((128, 256)),
            full((1, 128)),
            full((64, 128)),
            full((64, 128)),
            full((1, 64)),
            full((1, 64)),
            full((1, 1)),
        ],
        out_specs=pl.BlockSpec((_BLK, 1), lambda i: (i, 0)),
        out_shape=jax.ShapeDtypeStruct((_B, 1), jnp.float32),
    )(pooled, semantic, W_ph, b_ph, W_se, b_se, wc1, wc2, b_c, W_o, b_o)


def kernel(phonetic_token, semantic_embed, emb_table,
           W_ph, b_ph, W_se, b_se, W_c, b_c, W_o, b_o):
    pooled = _pool(phonetic_token.astype(jnp.int32), emb_table)
    return _mlp(pooled, semantic_embed,
                W_ph, b_ph.reshape(1, -1),
                W_se, b_se.reshape(1, -1),
                W_c[:, :128], W_c[:, 128:], b_c.reshape(1, -1),
                W_o, b_o.reshape(1, -1))


# speedup vs baseline: 1.2009x; 1.0042x over previous
import jax
import jax.numpy as jnp
from jax import lax
from jax.experimental import pallas as pl
from jax.experimental.pallas import tpu as pltpu
from jax.experimental.pallas import tpu_sc as plsc

_B = 16384
_L = 200
_EMB = 64
_VOCAB = 1000000
_NC, _NS = 2, 16
_NW = _NC * _NS
_ROWS_PER_W = _B // _NW
_CA, _CB = 96, 104
_G = 256
_GROUPS = _ROWS_PER_W // _G
_INV_L = 1.0 / _L


def _pool_body(idx_hbm, table_hbm, out_hbm, idx_a, idx_b,
               buf0, buf1, buf2, buf3, out_v, sem0, sem1, sem2, sem3):
    wid = lax.axis_index("s") * _NC + lax.axis_index("c")
    row0 = wid * _ROWS_PER_W

    def accum(buf, n):
        def body(j, accs):
            a0, a1, a2, a3 = accs
            a0 = a0 + buf[j, 0:16]
            a1 = a1 + buf[j, 16:32]
            a2 = a2 + buf[j, 32:48]
            a3 = a3 + buf[j, 48:64]
            return (a0, a1, a2, a3)
        z = jnp.zeros((16,), jnp.float32)
        return lax.fori_loop(0, n, body, (z, z, z, z), unroll=8)

    def group(g, _):
        r0 = row0 + g * _G
        pltpu.sync_copy(idx_hbm.at[pl.ds(r0, _G), pl.ds(0, _CA)], idx_a)
        pltpu.sync_copy(idx_hbm.at[pl.ds(r0, _G), pl.ds(_CA, _CB)], idx_b)
        pltpu.async_copy(table_hbm.at[idx_a.at[0]], buf0, sem0)
        pltpu.async_copy(table_hbm.at[idx_b.at[0]], buf1, sem1)
        pltpu.async_copy(table_hbm.at[idx_a.at[1]], buf2, sem2)
        pltpu.async_copy(table_hbm.at[idx_b.at[1]], buf3, sem3)

        def pair(p, _):
            ra = 2 * p
            rb = 2 * p + 1

            pltpu.make_async_copy(table_hbm.at[idx_a.at[0]], buf0, sem0).wait()
            a0, a1, a2, a3 = accum(buf0, _CA)

            @pl.when(ra + 2 < _G)
            def _():
                pltpu.async_copy(table_hbm.at[idx_a.at[ra + 2]], buf0, sem0)

            pltpu.make_async_copy(table_hbm.at[idx_b.at[0]], buf1, sem1).wait()
            b0, b1, b2, b3 = accum(buf1, _CB)
            out_v[ra, 0:16] = a0 + b0
            out_v[ra, 16:32] = a1 + b1
            out_v[ra, 32:48] = a2 + b2
            out_v[ra, 48:64] = a3 + b3

            @pl.when(ra + 2 < _G)
            def _():
                pltpu.async_copy(table_hbm.at[idx_b.at[ra + 2]], buf1, sem1)

            pltpu.make_async_copy(table_hbm.at[idx_a.at[0]], buf2, sem2).wait()
            a0, a1, a2, a3 = accum(buf2, _CA)

            @pl.when(rb + 2 < _G)
            def _():
                pltpu.async_copy(table_hbm.at[idx_a.at[rb + 2]], buf2, sem2)

            pltpu.make_async_copy(table_hbm.at[idx_b.at[0]], buf3, sem3).wait()
            b0, b1, b2, b3 = accum(buf3, _CB)
            out_v[rb, 0:16] = a0 + b0
            out_v[rb, 16:32] = a1 + b1
            out_v[rb, 32:48] = a2 + b2
            out_v[rb, 48:64] = a3 + b3

            @pl.when(rb + 2 < _G)
            def _():
                pltpu.async_copy(table_hbm.at[idx_b.at[rb + 2]], buf3, sem3)

            return 0

        lax.fori_loop(0, _G // 2, pair, 0)
        pltpu.sync_copy(out_v, out_hbm.at[pl.ds(r0, _G), :])
        return 0

    lax.fori_loop(0, _GROUPS, group, 0)


@jax.jit
def _pool(idx, table_hbm):
    mesh = plsc.VectorSubcoreMesh(core_axis_name="c", subcore_axis_name="s")
    return pl.kernel(
        _pool_body,
        out_type=jax.ShapeDtypeStruct((_B, _EMB), jnp.float32),
        mesh=mesh,
        compiler_params=pltpu.CompilerParams(use_tc_tiling_on_sc=False),
        scratch_types=[
            pltpu.VMEM((_G, _CA), jnp.int32),
            pltpu.VMEM((_G, _CB), jnp.int32),
            pltpu.VMEM((_CA, _EMB), jnp.float32),
            pltpu.VMEM((_CB, _EMB), jnp.float32),
            pltpu.VMEM((_CA, _EMB), jnp.float32),
            pltpu.VMEM((_CB, _EMB), jnp.float32),
            pltpu.VMEM((_G, _EMB), jnp.float32),
            pltpu.SemaphoreType.DMA,
            pltpu.SemaphoreType.DMA,
            pltpu.SemaphoreType.DMA,
            pltpu.SemaphoreType.DMA,
        ],
    )(idx, table_hbm)


_BLK = 256


def _mlp_body(pool_ref, sem_ref, wph_ref, bph_ref, wse_ref, bse_ref,
              wc1_ref, wc2_ref, bc_ref, wo_ref, bo_ref, out_ref):
    pool = pool_ref[...] * _INV_L
    dn = (((1,), (1,)), ((), ()))
    ph = lax.dot_general(pool, wph_ref[...], dn,
                         preferred_element_type=jnp.float32) + bph_ref[...]
    se = lax.dot_general(sem_ref[...], wse_ref[...], dn,
                         preferred_element_type=jnp.float32) + bse_ref[...]
    x = (lax.dot_general(ph, wc1_ref[...], dn,
                         preferred_element_type=jnp.float32)
         + lax.dot_general(se, wc2_ref[...], dn,
                           preferred_element_type=jnp.float32)
         + bc_ref[...])
    x = jnp.maximum(x, 0.0)
    o = jnp.sum(x * wo_ref[...], axis=1, keepdims=True) + bo_ref[...]
    out_ref[...] = jax.nn.sigmoid(o)


@jax.jit
def _mlp(pooled, semantic, W_ph, b_ph, W_se, b_se, wc1, wc2, b_c, W_o, b_o):
    n_blk = _B // _BLK
    full = lambda shape: pl.BlockSpec(shape, lambda i: (0, 0))
    return pl.pallas_call(
        _mlp_body,
        grid=(n_blk,),
        in_specs=[
            pl.BlockSpec((_BLK, _EMB), lambda i: (i, 0)),
            pl.BlockSpec((_BLK, 256), lambda i: (i, 0)),
            full((128, _EMB)),
            full((1, 128)),
            full((128, 256)),
            full((1, 128)),
            full((64, 128)),
            full((64, 128)),
            full((1, 64)),
            full((1, 64)),
            full((1, 1)),
        ],
        out_specs=pl.BlockSpec((_BLK, 1), lambda i: (i, 0)),
        out_shape=jax.ShapeDtypeStruct((_B, 1), jnp.float32),
    )(pooled, semantic, W_ph, b_ph, W_se, b_se, wc1, wc2, b_c, W_o, b_o)


def kernel(phonetic_token, semantic_embed, emb_table,
           W_ph, b_ph, W_se, b_se, W_c, b_c, W_o, b_o):
    pooled = _pool(phonetic_token.astype(jnp.int32), emb_table)
    return _mlp(pooled, semantic_embed,
                W_ph, b_ph.reshape(1, -1),
                W_se, b_se.reshape(1, -1),
                W_c[:, :128], W_c[:, 128:], b_c.reshape(1, -1),
                W_o, b_o.reshape(1, -1))


# speedup vs baseline: 1.2754x; 1.0621x over previous
import jax
import jax.numpy as jnp
from jax import lax
from jax.experimental import pallas as pl
from jax.experimental.pallas import tpu as pltpu
from jax.experimental.pallas import tpu_sc as plsc

_B = 16384
_L = 200
_EMB = 64
_VOCAB = 1000000
_NC, _NS = 2, 16
_NW = _NC * _NS
_ROWS_PER_W = _B // _NW
_CA, _CB = 96, 104
_G = 128
_GROUPS = _ROWS_PER_W // _G
_INV_L = 1.0 / _L


def _pool_body(idx_hbm, table_hbm, out_hbm, idx_a, idx_b,
               buf0, buf1, buf2, buf3, buf4, buf5, buf6, buf7, out_v,
               sem0, sem1, sem2, sem3, sem4, sem5, sem6, sem7):
    wid = lax.axis_index("s") * _NC + lax.axis_index("c")
    row0 = wid * _ROWS_PER_W

    def accum(buf, n):
        def body(j, accs):
            a0, a1, a2, a3 = accs
            a0 = a0 + buf[j, 0:16]
            a1 = a1 + buf[j, 16:32]
            a2 = a2 + buf[j, 32:48]
            a3 = a3 + buf[j, 48:64]
            return (a0, a1, a2, a3)
        z = jnp.zeros((16,), jnp.float32)
        return lax.fori_loop(0, n, body, (z, z, z, z), unroll=8)

    def group(g, _):
        r0 = row0 + g * _G
        pltpu.sync_copy(idx_hbm.at[pl.ds(r0, _G), pl.ds(0, _CA)], idx_a)
        pltpu.sync_copy(idx_hbm.at[pl.ds(r0, _G), pl.ds(_CA, _CB)], idx_b)
        abufs = (buf0, buf2, buf4, buf6)
        bbufs = (buf1, buf3, buf5, buf7)
        asems = (sem0, sem2, sem4, sem6)
        bsems = (sem1, sem3, sem5, sem7)
        for q in range(4):
            pltpu.async_copy(table_hbm.at[idx_a.at[q]], abufs[q], asems[q])
            pltpu.async_copy(table_hbm.at[idx_b.at[q]], bbufs[q], bsems[q])

        def quad(p, _):
            for q in range(4):
                r = 4 * p + q
                ba, sa = abufs[q], asems[q]
                bb, sb = bbufs[q], bsems[q]
                pltpu.make_async_copy(table_hbm.at[idx_a.at[0]], ba, sa).wait()
                a0, a1, a2, a3 = accum(ba, _CA)

                @pl.when(r + 4 < _G)
                def _(r=r, ba=ba, sa=sa):
                    pltpu.async_copy(table_hbm.at[idx_a.at[r + 4]], ba, sa)

                pltpu.make_async_copy(table_hbm.at[idx_b.at[0]], bb, sb).wait()
                b0, b1, b2, b3 = accum(bb, _CB)
                out_v[r, 0:16] = a0 + b0
                out_v[r, 16:32] = a1 + b1
                out_v[r, 32:48] = a2 + b2
                out_v[r, 48:64] = a3 + b3

                @pl.when(r + 4 < _G)
                def _(r=r, bb=bb, sb=sb):
                    pltpu.async_copy(table_hbm.at[idx_b.at[r + 4]], bb, sb)

            return 0

        lax.fori_loop(0, _G // 4, quad, 0)
        pltpu.sync_copy(out_v, out_hbm.at[pl.ds(r0, _G), :])
        return 0

    lax.fori_loop(0, _GROUPS, group, 0)


@jax.jit
def _pool(idx, table_hbm):
    mesh = plsc.VectorSubcoreMesh(core_axis_name="c", subcore_axis_name="s")
    return pl.kernel(
        _pool_body,
        out_type=jax.ShapeDtypeStruct((_B, _EMB), jnp.float32),
        mesh=mesh,
        compiler_params=pltpu.CompilerParams(use_tc_tiling_on_sc=False),
        scratch_types=[
            pltpu.VMEM((_G, _CA), jnp.int32),
            pltpu.VMEM((_G, _CB), jnp.int32),
            pltpu.VMEM((_CA, _EMB), jnp.float32),
            pltpu.VMEM((_CB, _EMB), jnp.float32),
            pltpu.VMEM((_CA, _EMB), jnp.float32),
            pltpu.VMEM((_CB, _EMB), jnp.float32),
            pltpu.VMEM((_CA, _EMB), jnp.float32),
            pltpu.VMEM((_CB, _EMB), jnp.float32),
            pltpu.VMEM((_CA, _EMB), jnp.float32),
            pltpu.VMEM((_CB, _EMB), jnp.float32),
            pltpu.VMEM((_G, _EMB), jnp.float32),
            pltpu.SemaphoreType.DMA,
            pltpu.SemaphoreType.DMA,
            pltpu.SemaphoreType.DMA,
            pltpu.SemaphoreType.DMA,
            pltpu.SemaphoreType.DMA,
            pltpu.SemaphoreType.DMA,
            pltpu.SemaphoreType.DMA,
            pltpu.SemaphoreType.DMA,
        ],
    )(idx, table_hbm)


_BLK = 256


def _mlp_body(pool_ref, sem_ref, wph_ref, bph_ref, wse_ref, bse_ref,
              wc1_ref, wc2_ref, bc_ref, wo_ref, bo_ref, out_ref):
    pool = pool_ref[...] * _INV_L
    dn = (((1,), (1,)), ((), ()))
    ph = lax.dot_general(pool, wph_ref[...], dn,
                         preferred_element_type=jnp.float32) + bph_ref[...]
    se = lax.dot_general(sem_ref[...], wse_ref[...], dn,
                         preferred_element_type=jnp.float32) + bse_ref[...]
    x = (lax.dot_general(ph, wc1_ref[...], dn,
                         preferred_element_type=jnp.float32)
         + lax.dot_general(se, wc2_ref[...], dn,
                           preferred_element_type=jnp.float32)
         + bc_ref[...])
    x = jnp.maximum(x, 0.0)
    o = jnp.sum(x * wo_ref[...], axis=1, keepdims=True) + bo_ref[...]
    out_ref[...] = jax.nn.sigmoid(o)


@jax.jit
def _mlp(pooled, semantic, W_ph, b_ph, W_se, b_se, wc1, wc2, b_c, W_o, b_o):
    n_blk = _B // _BLK
    full = lambda shape: pl.BlockSpec(shape, lambda i: (0, 0))
    return pl.pallas_call(
        _mlp_body,
        grid=(n_blk,),
        in_specs=[
            pl.BlockSpec((_BLK, _EMB), lambda i: (i, 0)),
            pl.BlockSpec((_BLK, 256), lambda i: (i, 0)),
            full((128, _EMB)),
            full((1, 128)),
            full((128, 256)),
            full((1, 128)),
            full((64, 128)),
            full((64, 128)),
            full((1, 64)),
            full((1, 64)),
            full((1, 1)),
        ],
        out_specs=pl.BlockSpec((_BLK, 1), lambda i: (i, 0)),
        out_shape=jax.ShapeDtypeStruct((_B, 1), jnp.float32),
    )(pooled, semantic, W_ph, b_ph, W_se, b_se, wc1, wc2, b_c, W_o, b_o)


def kernel(phonetic_token, semantic_embed, emb_table,
           W_ph, b_ph, W_se, b_se, W_c, b_c, W_o, b_o):
    pooled = _pool(phonetic_token.astype(jnp.int32), emb_table)
    return _mlp(pooled, semantic_embed,
                W_ph, b_ph.reshape(1, -1),
                W_se, b_se.reshape(1, -1),
                W_c[:, :128], W_c[:, 128:], b_c.reshape(1, -1),
                W_o, b_o.reshape(1, -1))


# speedup vs baseline: 1.2817x; 1.0049x over previous
import jax
import jax.numpy as jnp
from jax import lax
from jax.experimental import pallas as pl
from jax.experimental.pallas import tpu as pltpu
from jax.experimental.pallas import tpu_sc as plsc

_B = 16384
_L = 200
_EMB = 64
_VOCAB = 1000000
_NC, _NS = 2, 16
_NW = _NC * _NS
_ROWS_PER_W = _B // _NW
_CA, _CB = 96, 104
_G = 256
_GROUPS = _ROWS_PER_W // _G
_INV_L = 1.0 / _L


def _pool_body(idx_hbm, table_hbm, out_hbm, idx_a, idx_b,
               buf0, buf1, buf2, buf3, buf4, buf5, buf6, buf7, out_v,
               sem0, sem1, sem2, sem3, sem4, sem5, sem6, sem7):
    wid = lax.axis_index("s") * _NC + lax.axis_index("c")
    row0 = wid * _ROWS_PER_W

    def accum(buf, n):
        def body(j, accs):
            a0, a1, a2, a3 = accs
            a0 = a0 + buf[j, 0:16]
            a1 = a1 + buf[j, 16:32]
            a2 = a2 + buf[j, 32:48]
            a3 = a3 + buf[j, 48:64]
            return (a0, a1, a2, a3)
        z = jnp.zeros((16,), jnp.float32)
        return lax.fori_loop(0, n, body, (z, z, z, z), unroll=8)

    def group(g, _):
        r0 = row0 + g * _G
        pltpu.sync_copy(idx_hbm.at[pl.ds(r0, _G), pl.ds(0, _CA)], idx_a)
        pltpu.sync_copy(idx_hbm.at[pl.ds(r0, _G), pl.ds(_CA, _CB)], idx_b)
        abufs = (buf0, buf2, buf4, buf6)
        bbufs = (buf1, buf3, buf5, buf7)
        asems = (sem0, sem2, sem4, sem6)
        bsems = (sem1, sem3, sem5, sem7)
        for q in range(4):
            pltpu.async_copy(table_hbm.at[idx_a.at[q]], abufs[q], asems[q])
            pltpu.async_copy(table_hbm.at[idx_b.at[q]], bbufs[q], bsems[q])

        def quad(p, _):
            for q in range(4):
                r = 4 * p + q
                ba, sa = abufs[q], asems[q]
                bb, sb = bbufs[q], bsems[q]
                pltpu.make_async_copy(table_hbm.at[idx_a.at[0]], ba, sa).wait()
                a0, a1, a2, a3 = accum(ba, _CA)

                @pl.when(r + 4 < _G)
                def _(r=r, ba=ba, sa=sa):
                    pltpu.async_copy(table_hbm.at[idx_a.at[r + 4]], ba, sa)

                pltpu.make_async_copy(table_hbm.at[idx_b.at[0]], bb, sb).wait()
                b0, b1, b2, b3 = accum(bb, _CB)
                out_v[r, 0:16] = a0 + b0
                out_v[r, 16:32] = a1 + b1
                out_v[r, 32:48] = a2 + b2
                out_v[r, 48:64] = a3 + b3

                @pl.when(r + 4 < _G)
                def _(r=r, bb=bb, sb=sb):
                    pltpu.async_copy(table_hbm.at[idx_b.at[r + 4]], bb, sb)

            return 0

        lax.fori_loop(0, _G // 4, quad, 0)
        pltpu.sync_copy(out_v, out_hbm.at[pl.ds(r0, _G), :])
        return 0

    lax.fori_loop(0, _GROUPS, group, 0)


@jax.jit
def _pool(idx, table_hbm):
    mesh = plsc.VectorSubcoreMesh(core_axis_name="c", subcore_axis_name="s")
    return pl.kernel(
        _pool_body,
        out_type=jax.ShapeDtypeStruct((_B, _EMB), jnp.float32),
        mesh=mesh,
        compiler_params=pltpu.CompilerParams(use_tc_tiling_on_sc=False),
        scratch_types=[
            pltpu.VMEM((_G, _CA), jnp.int32),
            pltpu.VMEM((_G, _CB), jnp.int32),
            pltpu.VMEM((_CA, _EMB), jnp.float32),
            pltpu.VMEM((_CB, _EMB), jnp.float32),
            pltpu.VMEM((_CA, _EMB), jnp.float32),
            pltpu.VMEM((_CB, _EMB), jnp.float32),
            pltpu.VMEM((_CA, _EMB), jnp.float32),
            pltpu.VMEM((_CB, _EMB), jnp.float32),
            pltpu.VMEM((_CA, _EMB), jnp.float32),
            pltpu.VMEM((_CB, _EMB), jnp.float32),
            pltpu.VMEM((_G, _EMB), jnp.float32),
            pltpu.SemaphoreType.DMA,
            pltpu.SemaphoreType.DMA,
            pltpu.SemaphoreType.DMA,
            pltpu.SemaphoreType.DMA,
            pltpu.SemaphoreType.DMA,
            pltpu.SemaphoreType.DMA,
            pltpu.SemaphoreType.DMA,
            pltpu.SemaphoreType.DMA,
        ],
    )(idx, table_hbm)


_BLK = 256


def _mlp_body(pool_ref, sem_ref, wph_ref, bph_ref, wse_ref, bse_ref,
              wc1_ref, wc2_ref, bc_ref, wo_ref, bo_ref, out_ref):
    pool = pool_ref[...] * _INV_L
    dn = (((1,), (1,)), ((), ()))
    ph = lax.dot_general(pool, wph_ref[...], dn,
                         preferred_element_type=jnp.float32) + bph_ref[...]
    se = lax.dot_general(sem_ref[...], wse_ref[...], dn,
                         preferred_element_type=jnp.float32) + bse_ref[...]
    x = (lax.dot_general(ph, wc1_ref[...], dn,
                         preferred_element_type=jnp.float32)
         + lax.dot_general(se, wc2_ref[...], dn,
                           preferred_element_type=jnp.float32)
         + bc_ref[...])
    x = jnp.maximum(x, 0.0)
    o = jnp.sum(x * wo_ref[...], axis=1, keepdims=True) + bo_ref[...]
    out_ref[...] = jax.nn.sigmoid(o)


@jax.jit
def _mlp(pooled, semantic, W_ph, b_ph, W_se, b_se, wc1, wc2, b_c, W_o, b_o):
    n_blk = _B // _BLK
    full = lambda shape: pl.BlockSpec(shape, lambda i: (0, 0))
    return pl.pallas_call(
        _mlp_body,
        grid=(n_blk,),
        in_specs=[
            pl.BlockSpec((_BLK, _EMB), lambda i: (i, 0)),
            pl.BlockSpec((_BLK, 256), lambda i: (i, 0)),
            full((128, _EMB)),
            full((1, 128)),
            full((128, 256)),
            full((1, 128)),
            full((64, 128)),
            full((64, 128)),
            full((1, 64)),
            full((1, 64)),
            full((1, 1)),
        ],
        out_specs=pl.BlockSpec((_BLK, 1), lambda i: (i, 0)),
        out_shape=jax.ShapeDtypeStruct((_B, 1), jnp.float32),
    )(pooled, semantic, W_ph, b_ph, W_se, b_se, wc1, wc2, b_c, W_o, b_o)


def kernel(phonetic_token, semantic_embed, emb_table,
           W_ph, b_ph, W_se, b_se, W_c, b_c, W_o, b_o):
    pooled = _pool(phonetic_token.astype(jnp.int32), emb_table)
    return _mlp(pooled, semantic_embed,
                W_ph, b_ph.reshape(1, -1),
                W_se, b_se.reshape(1, -1),
                W_c[:, :128], W_c[:, 128:], b_c.reshape(1, -1),
                W_o, b_o.reshape(1, -1))
